# Initial kernel scaffold; baseline (speedup 1.0000x reference)
import jax
import jax.numpy as jnp
from jax.experimental import pallas as pl


def kernel(user_emb, item_emb, adj_vals, adj_rows, adj_cols, users, items):
    raise NotImplementedError("write your pallas kernel here")


# speedup vs baseline: 5.2616x; 5.2616x over previous
import functools

import jax
import jax.numpy as jnp
from jax import lax
from jax.experimental import pallas as pl
from jax.experimental.pallas import tpu as pltpu
from jax.experimental.pallas import tpu_sc as plsc

USERS = 60000
ITEMS = 40000
N = USERS + ITEMS
E = 1600000
EMB = 32
HE = EMB // 2
LAYERS = 3
ALPHA = 0.1
BETA = 0.9
GAMMA = BETA ** LAYERS + ALPHA * sum(BETA ** i for i in range(LAYERS))
BATCH = 4096

NC = 2
NS = 16
SUB = 128
K = 512
NSUB = K // SUB
CHUNKS = E // K
CPT = -(-CHUNKS // NS)
RB = 128
ACC_ROWS = 100096
ZB = ACC_ROWS // RB
ZBPT = -(-ZB // NS)
RFULL = N // RB
RTAIL = N - RFULL * RB
RBPT = -(-(RFULL + 1) // NS)
GPT = BATCH // (NC * NS)

_mesh = functools.partial(plsc.VectorSubcoreMesh,
                          core_axis_name="c", subcore_axis_name="s")
_params = pltpu.CompilerParams(needs_layout_passes=False,
                               use_tc_tiling_on_sc=False)


def _iota16():
    return lax.iota(jnp.int32, 16)


def _make_layer(first: bool, beta: float, alpha: float):
    scratch = [
        pltpu.VMEM_SHARED((ACC_ROWS, HE), jnp.float32),
        pltpu.VMEM((NSUB, SUB), jnp.int32),
        pltpu.VMEM((NSUB, SUB), jnp.int32),
        pltpu.VMEM((NSUB, SUB), jnp.int32),
        pltpu.VMEM((K,), jnp.float32),
        pltpu.VMEM((K, HE), jnp.float32),
        pltpu.VMEM((RB, HE), jnp.float32),
        pltpu.VMEM((RB, HE), jnp.float32),
        pltpu.SemaphoreType.DMA,
    ]

    def body(*refs):
        if first:
            (ego, cols2, rows2, vals, out,
             acc, cols_v, rows_v, idx2, vals_v, gath, obuf, xbuf, sem) = refs
            x0 = None
        else:
            (ego, cols2, rows2, vals, x0, out,
             acc, cols_v, rows_v, idx2, vals_v, gath, obuf, xbuf, sem) = refs
        c = lax.axis_index("c")
        s = lax.axis_index("s")
        cN = c * N

        def zrow(r, _):
            obuf[r, pl.ds(0, HE)] = jnp.zeros((HE,), jnp.float32)
            return 0
        lax.fori_loop(0, RB, zrow, 0)

        def zblk(i, _):
            ch = s + NS * i
            @pl.when(ch < ZB)
            def _():
                pltpu.sync_copy(obuf, acc.at[pl.ds(ch * RB, RB)])
            return 0
        lax.fori_loop(0, ZBPT, zblk, 0)
        plsc.subcore_barrier()

        def chunk(i, _):
            ch = s + NS * i

            @pl.when(ch < CHUNKS)
            def _():
                pltpu.sync_copy(cols2.at[pl.ds(ch * NSUB, NSUB)], cols_v)
                pltpu.sync_copy(rows2.at[pl.ds(ch * NSUB, NSUB)], rows_v)
                pltpu.sync_copy(vals.at[pl.ds(ch * K, K)], vals_v)

                for g in range(K // 16):
                    cv = cols_v[(g * 16) // SUB, pl.ds((g * 16) % SUB, 16)]
                    hidx = cv * 2 + c if first else cv + cN
                    idx2[(g * 16) // SUB, pl.ds((g * 16) % SUB, 16)] = hidx

                copies = [
                    pltpu.async_copy(ego.at[idx2.at[j]],
                                     gath.at[pl.ds(j * SUB, SUB)], sem)
                    for j in range(NSUB)
                ]
                for cp in copies:
                    cp.wait()

                def scale(g, _):
                    v16 = vals_v[pl.ds(g * 16, 16)]
                    e16 = g * 16 + _iota16()
                    for j in range(HE):
                        j16 = jnp.full((16,), j, jnp.int32)
                        x = plsc.load_gather(gath, [e16, j16])
                        plsc.store_scatter(gath, [e16, j16], x * v16)
                    return 0
                lax.fori_loop(0, K // 16, scale, 0)

                for j in range(NSUB):
                    pltpu.sync_copy(gath.at[pl.ds(j * SUB, SUB)],
                                    acc.at[rows_v.at[j]], add=True)
            return 0
        lax.fori_loop(0, CPT, chunk, 0)
        plsc.subcore_barrier()

        def emit(r0, nrows):
            pltpu.sync_copy(acc.at[pl.ds(r0, nrows)], obuf.at[pl.ds(0, nrows)])
            if not first:
                pltpu.sync_copy(x0.at[pl.ds(cN + r0, nrows)],
                                xbuf.at[pl.ds(0, nrows)])

                def mrow(r, _):
                    obuf[r, pl.ds(0, HE)] = (obuf[r, pl.ds(0, HE)] * beta
                                             + xbuf[r, pl.ds(0, HE)] * alpha)
                    return 0
                lax.fori_loop(0, nrows, mrow, 0)
            pltpu.sync_copy(obuf.at[pl.ds(0, nrows)],
                            out.at[pl.ds(cN + r0, nrows)])

        def eblk(i, _):
            ch = s + NS * i
            @pl.when(ch < RFULL)
            def _():
                emit(ch * RB, RB)
            return 0
        lax.fori_loop(0, RBPT, eblk, 0)
        @pl.when(s == (RFULL % NS))
        def _():
            emit(RFULL * RB, RTAIL)

    return pl.kernel(body,
                     out_type=jax.ShapeDtypeStruct((NC * N, HE), jnp.float32),
                     mesh=_mesh(),
                     scratch_types=scratch,
                     compiler_params=_params)


_layer_first = _make_layer(True, 1.0, 0.0)
_layer_mid = _make_layer(False, BETA, ALPHA)
_layer_last = _make_layer(False, BETA / GAMMA, ALPHA / GAMMA)


def _lookup_body(ego, users, items, u_out, i_out, idx_v, rbuf, sem):
    c = lax.axis_index("c")
    s = lax.axis_index("s")
    wid = s * NC + c
    off = wid * GPT
    for src, node0, half, dst in ((users, 0, 0, u_out), (users, 0, N, u_out),
                                  (items, USERS, 0, i_out),
                                  (items, USERS, N, i_out)):
        pltpu.sync_copy(src.at[pl.ds(off, GPT)], idx_v)
        shift = node0 + half
        if shift:
            for g in range(GPT // 16):
                idx_v[pl.ds(g * 16, 16)] = idx_v[pl.ds(g * 16, 16)] + shift
        pltpu.async_copy(ego.at[idx_v], rbuf, sem).wait()
        obase = (BATCH if half else 0) + off
        pltpu.sync_copy(rbuf, dst.at[pl.ds(obase, GPT)])


_lookup = pl.kernel(
    _lookup_body,
    out_type=(jax.ShapeDtypeStruct((NC * BATCH, HE), jnp.float32),
              jax.ShapeDtypeStruct((NC * BATCH, HE), jnp.float32)),
    mesh=_mesh(),
    scratch_types=[
        pltpu.VMEM((GPT,), jnp.int32),
        pltpu.VMEM((GPT, HE), jnp.float32),
        pltpu.SemaphoreType.DMA,
    ],
    compiler_params=_params)


def kernel(user_emb, item_emb, adj_vals, adj_rows, adj_cols, users, items):
    ego0 = jnp.concatenate([user_emb, item_emb], axis=0).reshape(NC * N, HE)
    cols2 = adj_cols.reshape(E // SUB, SUB)
    rows2 = adj_rows.reshape(E // SUB, SUB)
    h1 = _layer_first(ego0, cols2, rows2, adj_vals)
    h2 = _layer_mid(h1, cols2, rows2, adj_vals, h1)
    h3 = _layer_last(h2, cols2, rows2, adj_vals, h1)
    u2, i2 = _lookup(h3, users, items)
    u = jnp.concatenate([u2[:BATCH], u2[BATCH:]], axis=1)
    it = jnp.concatenate([i2[:BATCH], i2[BATCH:]], axis=1)
    return (u, it)


# speedup vs baseline: 7.8399x; 1.4900x over previous
import functools

import jax
import jax.numpy as jnp
from jax import lax
from jax.experimental import pallas as pl
from jax.experimental.pallas import tpu as pltpu
from jax.experimental.pallas import tpu_sc as plsc

USERS = 60000
ITEMS = 40000
N = USERS + ITEMS
E = 1600000
EMB = 32
HE = EMB // 2
LAYERS = 3
ALPHA = 0.1
BETA = 0.9
GAMMA = BETA ** LAYERS + ALPHA * sum(BETA ** i for i in range(LAYERS))
BATCH = 4096

NC = 2
NS = 16
SUB = 128
K = 512
NSUB = K // SUB
CHUNKS = E // K
CPT = -(-CHUNKS // NS)
RB = 128
ACC_ROWS = 100096
ZB = ACC_ROWS // RB
ZBPT = -(-ZB // NS)
RFULL = N // RB
RTAIL = N - RFULL * RB
RBPT = -(-(RFULL + 1) // NS)
GPT = BATCH // (NC * NS)

assert CPT % 2 == 0

_mesh = functools.partial(plsc.VectorSubcoreMesh,
                          core_axis_name="c", subcore_axis_name="s")
_params = pltpu.CompilerParams(needs_layout_passes=False,
                               use_tc_tiling_on_sc=False)


def _iota16():
    return lax.iota(jnp.int32, 16)


def _make_layer(first: bool, beta: float, alpha: float):
    scratch = [
        pltpu.VMEM_SHARED((ACC_ROWS, HE), jnp.float32),
        pltpu.VMEM((NSUB, 3, SUB), jnp.int32),
        [pltpu.VMEM((NSUB, SUB), jnp.int32)] * 2,
        [pltpu.VMEM((NSUB, SUB), jnp.int32)] * 2,
        [pltpu.VMEM((K,), jnp.float32)] * 2,
        [pltpu.VMEM((K, HE), jnp.float32)] * 2,
        pltpu.VMEM((RB, HE), jnp.float32),
        pltpu.VMEM((RB, HE), jnp.float32),
        pltpu.SemaphoreType.DMA,
        [pltpu.SemaphoreType.DMA] * 2,
        [pltpu.SemaphoreType.DMA] * 2,
    ]

    def body(*refs):
        if first:
            (ego, edges3, out, acc, ebuf, rows_v, idx2, vals_v, gath,
             obuf, xbuf, esem, gsem, ssem) = refs
            x0 = None
        else:
            (ego, edges3, x0, out, acc, ebuf, rows_v, idx2, vals_v, gath,
             obuf, xbuf, esem, gsem, ssem) = refs
        c = lax.axis_index("c")
        s = lax.axis_index("s")
        cN = c * N

        def zrow(r, _):
            obuf[r, pl.ds(0, HE)] = jnp.zeros((HE,), jnp.float32)
            return 0
        lax.fori_loop(0, RB, zrow, 0)

        def zblk(i, _):
            ch = s + NS * i
            @pl.when(ch < ZB)
            def _():
                pltpu.sync_copy(obuf, acc.at[pl.ds(ch * RB, RB)])
            return 0
        lax.fori_loop(0, ZBPT, zblk, 0)
        plsc.subcore_barrier()

        def trip_chunk(i):
            return s + NS * i

        def edge_load(i):
            @pl.when(trip_chunk(i) < CHUNKS)
            def _():
                pltpu.async_copy(
                    edges3.at[pl.ds(trip_chunk(i) * NSUB, NSUB)], ebuf, esem)

        def prep(i, b):
            @pl.when(trip_chunk(i) < CHUNKS)
            def _():
                pltpu.make_async_copy(
                    edges3.at[pl.ds(0, NSUB)], ebuf, esem).wait()
                for g in range(K // 16):
                    j, o = (g * 16) // SUB, (g * 16) % SUB
                    cv = ebuf[j, 0, pl.ds(o, 16)]
                    idx2[b][j, pl.ds(o, 16)] = (cv * 2 + c if first
                                                else cv + cN)
                    rows_v[b][j, pl.ds(o, 16)] = ebuf[j, 1, pl.ds(o, 16)]
                    vals_v[b][pl.ds(g * 16, 16)] = plsc.bitcast(
                        ebuf[j, 2, pl.ds(o, 16)], jnp.float32)
                for j in range(NSUB):
                    pltpu.async_copy(ego.at[idx2[b].at[j]],
                                     gath[b].at[pl.ds(j * SUB, SUB)], gsem[b])

        def proc(i, b):
            @pl.when(trip_chunk(i) < CHUNKS)
            def _():
                for j in range(NSUB):
                    pltpu.make_async_copy(
                        ego.at[idx2[b].at[j]],
                        gath[b].at[pl.ds(j * SUB, SUB)], gsem[b]).wait()

                def scale(g, _):
                    v16 = vals_v[b][pl.ds(g * 16, 16)]
                    e16 = g * 16 + _iota16()
                    for j in range(HE):
                        j16 = jnp.full((16,), j, jnp.int32)
                        x = plsc.load_gather(gath[b], [e16, j16])
                        plsc.store_scatter(gath[b], [e16, j16], x * v16)
                    return 0
                lax.fori_loop(0, K // 16, scale, 0)

                for j in range(NSUB):
                    pltpu.async_copy(gath[b].at[pl.ds(j * SUB, SUB)],
                                     acc.at[rows_v[b].at[j]], ssem[b],
                                     add=True)

        def drain(cond, b):
            @pl.when(cond)
            def _():
                for j in range(NSUB):
                    pltpu.make_async_copy(gath[b].at[pl.ds(j * SUB, SUB)],
                                          acc.at[rows_v[b].at[j]],
                                          ssem[b]).wait()

        edge_load(0)
        prep(0, 0)
        edge_load(1)

        def outer(io, _):
            for b in (0, 1):
                i = io * 2 + b
                drain(jnp.logical_and(i >= 1, trip_chunk(i - 1) < CHUNKS),
                      1 - b)
                prep(i + 1, 1 - b)
                edge_load(i + 2)
                proc(i, b)
            return 0
        lax.fori_loop(0, CPT // 2, outer, 0)
        drain(trip_chunk(CPT - 1) < CHUNKS, (CPT - 1) % 2)
        plsc.subcore_barrier()

        def emit(r0, nrows):
            pltpu.sync_copy(acc.at[pl.ds(r0, nrows)], obuf.at[pl.ds(0, nrows)])
            if not first:
                pltpu.sync_copy(x0.at[pl.ds(cN + r0, nrows)],
                                xbuf.at[pl.ds(0, nrows)])

                def mrow(r, _):
                    obuf[r, pl.ds(0, HE)] = (obuf[r, pl.ds(0, HE)] * beta
                                             + xbuf[r, pl.ds(0, HE)] * alpha)
                    return 0
                lax.fori_loop(0, nrows, mrow, 0)
            pltpu.sync_copy(obuf.at[pl.ds(0, nrows)],
                            out.at[pl.ds(cN + r0, nrows)])

        def eblk(i, _):
            ch = s + NS * i
            @pl.when(ch < RFULL)
            def _():
                emit(ch * RB, RB)
            return 0
        lax.fori_loop(0, RBPT, eblk, 0)
        @pl.when(s == (RFULL % NS))
        def _():
            emit(RFULL * RB, RTAIL)

    return pl.kernel(body,
                     out_type=jax.ShapeDtypeStruct((NC * N, HE), jnp.float32),
                     mesh=_mesh(),
                     scratch_types=scratch,
                     compiler_params=_params)


_layer_first = _make_layer(True, 1.0, 0.0)
_layer_mid = _make_layer(False, BETA, ALPHA)
_layer_last = _make_layer(False, BETA / GAMMA, ALPHA / GAMMA)


def _lookup_body(ego, users, items, u_out, i_out, idx_v, rbuf, sem):
    c = lax.axis_index("c")
    s = lax.axis_index("s")
    wid = s * NC + c
    off = wid * GPT
    for src, shift, dst, obase in (
            (users, 0, u_out, 0), (users, N, u_out, BATCH),
            (items, USERS, i_out, 0), (items, USERS + N, i_out, BATCH)):
        pltpu.sync_copy(src.at[pl.ds(off, GPT)], idx_v)
        if shift:
            for g in range(GPT // 16):
                idx_v[pl.ds(g * 16, 16)] = idx_v[pl.ds(g * 16, 16)] + shift
        pltpu.async_copy(ego.at[idx_v], rbuf, sem).wait()
        pltpu.sync_copy(rbuf, dst.at[pl.ds(obase + off, GPT)])


_lookup = pl.kernel(
    _lookup_body,
    out_type=(jax.ShapeDtypeStruct((NC * BATCH, HE), jnp.float32),
              jax.ShapeDtypeStruct((NC * BATCH, HE), jnp.float32)),
    mesh=_mesh(),
    scratch_types=[
        pltpu.VMEM((GPT,), jnp.int32),
        pltpu.VMEM((GPT, HE), jnp.float32),
        pltpu.SemaphoreType.DMA,
    ],
    compiler_params=_params)


def kernel(user_emb, item_emb, adj_vals, adj_rows, adj_cols, users, items):
    ego0 = jnp.concatenate([user_emb, item_emb], axis=0).reshape(NC * N, HE)
    vbits = lax.bitcast_convert_type(adj_vals, jnp.int32)
    edges3 = jnp.stack([adj_cols.reshape(E // SUB, SUB),
                        adj_rows.reshape(E // SUB, SUB),
                        vbits.reshape(E // SUB, SUB)], axis=1)
    h1 = _layer_first(ego0, edges3)
    h2 = _layer_mid(h1, edges3, h1)
    h3 = _layer_last(h2, edges3, h1)
    u2, i2 = _lookup(h3, users, items)
    u = jnp.concatenate([u2[:BATCH], u2[BATCH:]], axis=1)
    it = jnp.concatenate([i2[:BATCH], i2[BATCH:]], axis=1)
    return (u, it)


# speedup vs baseline: 17.8475x; 2.2765x over previous
import functools

import jax
import jax.numpy as jnp
from jax import lax
from jax.experimental import pallas as pl
from jax.experimental.pallas import tpu as pltpu
from jax.experimental.pallas import tpu_sc as plsc

USERS = 60000
ITEMS = 40000
N = USERS + ITEMS
E = 1600000
EMB = 32
HE = EMB // 2
LAYERS = 3
ALPHA = 0.1
BETA = 0.9
GAMMA = BETA ** LAYERS + ALPHA * sum(BETA ** i for i in range(LAYERS))
BATCH = 4096

NC = 2
NS = 16
SUB = 128
K = 512
NSUB = K // SUB
CHUNKS = E // K
CPT = -(-CHUNKS // NS)
RB = 128
ACC_ROWS = 100096
ZB = ACC_ROWS // RB
ZBPT = -(-ZB // NS)
RFULL = N // RB
RTAIL = N - RFULL * RB
RBPT = -(-(RFULL + 1) // NS)
GPT = BATCH // (NC * NS)

assert CPT % 2 == 0

_mesh = functools.partial(plsc.VectorSubcoreMesh,
                          core_axis_name="c", subcore_axis_name="s")
_params = pltpu.CompilerParams(needs_layout_passes=False,
                               use_tc_tiling_on_sc=False)


def _iota16():
    return lax.iota(jnp.int32, 16)


def _make_layer(first: bool, beta: float, alpha: float):
    scratch = [
        pltpu.VMEM_SHARED((ACC_ROWS, HE), jnp.float32),
        pltpu.VMEM((NSUB, 3, SUB), jnp.int32),
        [pltpu.VMEM((NSUB, SUB), jnp.int32)] * 2,
        [pltpu.VMEM((NSUB, SUB), jnp.int32)] * 2,
        [pltpu.VMEM((K,), jnp.float32)] * 2,
        [pltpu.VMEM((K, HE), jnp.float32)] * 2,
        pltpu.VMEM((RB, HE), jnp.float32),
        pltpu.VMEM((RB, HE), jnp.float32),
        pltpu.SemaphoreType.DMA,
        [pltpu.SemaphoreType.DMA] * 2,
        [pltpu.SemaphoreType.DMA] * 2,
    ]

    def body(*refs):
        if first:
            (ego, edges3, out, acc, ebuf, rows_v, idx2, vals_v, gath,
             obuf, xbuf, esem, gsem, ssem) = refs
            x0 = None
        else:
            (ego, edges3, x0, out, acc, ebuf, rows_v, idx2, vals_v, gath,
             obuf, xbuf, esem, gsem, ssem) = refs
        c = lax.axis_index("c")
        s = lax.axis_index("s")
        cN = c * N

        def zrow(r, _):
            obuf[r, pl.ds(0, HE)] = jnp.zeros((HE,), jnp.float32)
            return 0
        lax.fori_loop(0, RB, zrow, 0)

        def zblk(i, _):
            ch = s + NS * i
            @pl.when(ch < ZB)
            def _():
                pltpu.sync_copy(obuf, acc.at[pl.ds(ch * RB, RB)])
            return 0
        lax.fori_loop(0, ZBPT, zblk, 0)
        plsc.subcore_barrier()

        def trip_chunk(i):
            return s + NS * i

        def edge_load(i):
            @pl.when(trip_chunk(i) < CHUNKS)
            def _():
                pltpu.async_copy(
                    edges3.at[pl.ds(trip_chunk(i) * NSUB, NSUB)], ebuf, esem)

        def prep(i, b):
            @pl.when(trip_chunk(i) < CHUNKS)
            def _():
                pltpu.make_async_copy(
                    edges3.at[pl.ds(0, NSUB)], ebuf, esem).wait()
                for g in range(K // 16):
                    j, o = (g * 16) // SUB, (g * 16) % SUB
                    cv = ebuf[j, 0, pl.ds(o, 16)]
                    idx2[b][j, pl.ds(o, 16)] = (cv * 2 + c if first
                                                else cv + cN)
                    rows_v[b][j, pl.ds(o, 16)] = ebuf[j, 1, pl.ds(o, 16)]
                    vals_v[b][pl.ds(g * 16, 16)] = plsc.bitcast(
                        ebuf[j, 2, pl.ds(o, 16)], jnp.float32)
                for j in range(NSUB):
                    pltpu.async_copy(ego.at[idx2[b].at[j]],
                                     gath[b].at[pl.ds(j * SUB, SUB)], gsem[b])

        def proc(i, b):
            @pl.when(trip_chunk(i) < CHUNKS)
            def _():
                for j in range(NSUB):
                    pltpu.make_async_copy(
                        ego.at[idx2[b].at[j]],
                        gath[b].at[pl.ds(j * SUB, SUB)], gsem[b]).wait()

                def scale(g, _):
                    v16 = vals_v[b][pl.ds(g * 16, 16)]
                    e16 = g * 16 + _iota16()
                    cols16 = [jnp.full((16,), j, jnp.int32) for j in range(HE)]
                    xs = [plsc.load_gather(gath[b], [e16, cols16[j]])
                          for j in range(HE)]
                    for j in range(HE):
                        plsc.store_scatter(gath[b], [e16, cols16[j]],
                                           xs[j] * v16)
                    return 0
                lax.fori_loop(0, K // 16, scale, 0)

                for j in range(NSUB):
                    pltpu.async_copy(gath[b].at[pl.ds(j * SUB, SUB)],
                                     acc.at[rows_v[b].at[j]], ssem[b],
                                     add=True)

        def drain(cond, b):
            @pl.when(cond)
            def _():
                for j in range(NSUB):
                    pltpu.make_async_copy(gath[b].at[pl.ds(j * SUB, SUB)],
                                          acc.at[rows_v[b].at[j]],
                                          ssem[b]).wait()

        edge_load(0)
        prep(0, 0)
        edge_load(1)

        def outer(io, _):
            for b in (0, 1):
                i = io * 2 + b
                drain(jnp.logical_and(i >= 1, trip_chunk(i - 1) < CHUNKS),
                      1 - b)
                prep(i + 1, 1 - b)
                edge_load(i + 2)
                proc(i, b)
            return 0
        lax.fori_loop(0, CPT // 2, outer, 0)
        drain(trip_chunk(CPT - 1) < CHUNKS, (CPT - 1) % 2)
        plsc.subcore_barrier()

        def emit(r0, nrows):
            pltpu.sync_copy(acc.at[pl.ds(r0, nrows)], obuf.at[pl.ds(0, nrows)])
            if not first:
                pltpu.sync_copy(x0.at[pl.ds(cN + r0, nrows)],
                                xbuf.at[pl.ds(0, nrows)])

                def mrow(r, _):
                    obuf[r, pl.ds(0, HE)] = (obuf[r, pl.ds(0, HE)] * beta
                                             + xbuf[r, pl.ds(0, HE)] * alpha)
                    return 0
                lax.fori_loop(0, nrows, mrow, 0)
            pltpu.sync_copy(obuf.at[pl.ds(0, nrows)],
                            out.at[pl.ds(cN + r0, nrows)])

        def eblk(i, _):
            ch = s + NS * i
            @pl.when(ch < RFULL)
            def _():
                emit(ch * RB, RB)
            return 0
        lax.fori_loop(0, RBPT, eblk, 0)
        @pl.when(s == (RFULL % NS))
        def _():
            emit(RFULL * RB, RTAIL)

    return pl.kernel(body,
                     out_type=jax.ShapeDtypeStruct((NC * N, HE), jnp.float32),
                     mesh=_mesh(),
                     scratch_types=scratch,
                     compiler_params=_params)


_layer_first = _make_layer(True, 1.0, 0.0)
_layer_mid = _make_layer(False, BETA, ALPHA)
_layer_last = _make_layer(False, BETA / GAMMA, ALPHA / GAMMA)


def _lookup_body(ego, users, items, u_out, i_out, idx_v, rbuf, sem):
    c = lax.axis_index("c")
    s = lax.axis_index("s")
    wid = s * NC + c
    off = wid * GPT
    for src, shift, dst, obase in (
            (users, 0, u_out, 0), (users, N, u_out, BATCH),
            (items, USERS, i_out, 0), (items, USERS + N, i_out, BATCH)):
        pltpu.sync_copy(src.at[pl.ds(off, GPT)], idx_v)
        if shift:
            for g in range(GPT // 16):
                idx_v[pl.ds(g * 16, 16)] = idx_v[pl.ds(g * 16, 16)] + shift
        pltpu.async_copy(ego.at[idx_v], rbuf, sem).wait()
        pltpu.sync_copy(rbuf, dst.at[pl.ds(obase + off, GPT)])


_lookup = pl.kernel(
    _lookup_body,
    out_type=(jax.ShapeDtypeStruct((NC * BATCH, HE), jnp.float32),
              jax.ShapeDtypeStruct((NC * BATCH, HE), jnp.float32)),
    mesh=_mesh(),
    scratch_types=[
        pltpu.VMEM((GPT,), jnp.int32),
        pltpu.VMEM((GPT, HE), jnp.float32),
        pltpu.SemaphoreType.DMA,
    ],
    compiler_params=_params)


def kernel(user_emb, item_emb, adj_vals, adj_rows, adj_cols, users, items):
    ego0 = jnp.concatenate([user_emb, item_emb], axis=0).reshape(NC * N, HE)
    vbits = lax.bitcast_convert_type(adj_vals, jnp.int32)
    edges3 = jnp.stack([adj_cols.reshape(E // SUB, SUB),
                        adj_rows.reshape(E // SUB, SUB),
                        vbits.reshape(E // SUB, SUB)], axis=1)
    h1 = _layer_first(ego0, edges3)
    h2 = _layer_mid(h1, edges3, h1)
    h3 = _layer_last(h2, edges3, h1)
    u2, i2 = _lookup(h3, users, items)
    u = jnp.concatenate([u2[:BATCH], u2[BATCH:]], axis=1)
    it = jnp.concatenate([i2[:BATCH], i2[BATCH:]], axis=1)
    return (u, it)


# speedup vs baseline: 18.3364x; 1.0274x over previous
import functools

import jax
import jax.numpy as jnp
from jax import lax
from jax.experimental import pallas as pl
from jax.experimental.pallas import tpu as pltpu
from jax.experimental.pallas import tpu_sc as plsc

USERS = 60000
ITEMS = 40000
N = USERS + ITEMS
E = 1600000
EMB = 32
HE = EMB // 2
LAYERS = 3
ALPHA = 0.1
BETA = 0.9
GAMMA = BETA ** LAYERS + ALPHA * sum(BETA ** i for i in range(LAYERS))
BATCH = 4096

NC = 2
NS = 16
SUB = 128
K = 512
NSUB = K // SUB
CHUNKS = E // K
CPT = -(-CHUNKS // NS)
RB = 256
ACC_ROWS = 100096
ZB = ACC_ROWS // RB
ZBPT = -(-ZB // NS)
RFULL = N // RB
RTAIL = N - RFULL * RB
RBPT = -(-(RFULL + 1) // NS)
GPT = BATCH // (NC * NS)

assert CPT % 2 == 0

_mesh = functools.partial(plsc.VectorSubcoreMesh,
                          core_axis_name="c", subcore_axis_name="s")
_params = pltpu.CompilerParams(needs_layout_passes=False,
                               use_tc_tiling_on_sc=False)


def _iota16():
    return lax.iota(jnp.int32, 16)


def _make_layer(first: bool, beta: float, alpha: float):
    scratch = [
        pltpu.VMEM_SHARED((ACC_ROWS, HE), jnp.float32),
        pltpu.VMEM((NSUB, 3, SUB), jnp.int32),
        [pltpu.VMEM((NSUB, SUB), jnp.int32)] * 2,
        [pltpu.VMEM((NSUB, SUB), jnp.int32)] * 2,
        [pltpu.VMEM((K,), jnp.float32)] * 2,
        [pltpu.VMEM((K, HE), jnp.float32)] * 2,
        pltpu.VMEM((RB, HE), jnp.float32),
        pltpu.VMEM((RB, HE), jnp.float32),
        pltpu.SemaphoreType.DMA,
        [pltpu.SemaphoreType.DMA] * 2,
        [pltpu.SemaphoreType.DMA] * 2,
    ]

    def body(*refs):
        if first:
            (ego, edges3, out, acc, ebuf, rows_v, idx2, vals_v, gath,
             obuf, xbuf, esem, gsem, ssem) = refs
            x0 = None
        else:
            (ego, edges3, x0, out, acc, ebuf, rows_v, idx2, vals_v, gath,
             obuf, xbuf, esem, gsem, ssem) = refs
        c = lax.axis_index("c")
        s = lax.axis_index("s")
        cN = c * N

        def zrow(r, _):
            obuf[r, pl.ds(0, HE)] = jnp.zeros((HE,), jnp.float32)
            return 0
        lax.fori_loop(0, RB, zrow, 0)

        def zblk(i, _):
            ch = s + NS * i
            @pl.when(ch < ZB)
            def _():
                pltpu.sync_copy(obuf, acc.at[pl.ds(ch * RB, RB)])
            return 0
        lax.fori_loop(0, ZBPT, zblk, 0)
        plsc.subcore_barrier()

        def trip_chunk(i):
            return s + NS * i

        def edge_load(i):
            @pl.when(trip_chunk(i) < CHUNKS)
            def _():
                pltpu.async_copy(
                    edges3.at[pl.ds(trip_chunk(i) * NSUB, NSUB)], ebuf, esem)

        def prep(i, b):
            @pl.when(trip_chunk(i) < CHUNKS)
            def _():
                pltpu.make_async_copy(
                    edges3.at[pl.ds(0, NSUB)], ebuf, esem).wait()
                for g in range(K // 16):
                    j, o = (g * 16) // SUB, (g * 16) % SUB
                    cv = ebuf[j, 0, pl.ds(o, 16)]
                    idx2[b][j, pl.ds(o, 16)] = (cv * 2 + c if first
                                                else cv + cN)
                    rows_v[b][j, pl.ds(o, 16)] = ebuf[j, 1, pl.ds(o, 16)]
                    vals_v[b][pl.ds(g * 16, 16)] = plsc.bitcast(
                        ebuf[j, 2, pl.ds(o, 16)], jnp.float32)
                for j in range(NSUB):
                    pltpu.async_copy(ego.at[idx2[b].at[j]],
                                     gath[b].at[pl.ds(j * SUB, SUB)], gsem[b])

        def proc(i, b):
            @pl.when(trip_chunk(i) < CHUNKS)
            def _():
                for j in range(NSUB):
                    pltpu.make_async_copy(
                        ego.at[idx2[b].at[j]],
                        gath[b].at[pl.ds(j * SUB, SUB)], gsem[b]).wait()

                def scale(g, _):
                    v16 = vals_v[b][pl.ds(g * 16, 16)]
                    e16 = g * 16 + _iota16()
                    cols16 = [jnp.full((16,), j, jnp.int32) for j in range(HE)]
                    xs = [plsc.load_gather(gath[b], [e16, cols16[j]])
                          for j in range(HE)]
                    for j in range(HE):
                        plsc.store_scatter(gath[b], [e16, cols16[j]],
                                           xs[j] * v16)
                    return 0
                lax.fori_loop(0, K // 16, scale, 0)

                for j in range(NSUB):
                    pltpu.async_copy(gath[b].at[pl.ds(j * SUB, SUB)],
                                     acc.at[rows_v[b].at[j]], ssem[b],
                                     add=True)

        def drain(cond, b):
            @pl.when(cond)
            def _():
                for j in range(NSUB):
                    pltpu.make_async_copy(gath[b].at[pl.ds(j * SUB, SUB)],
                                          acc.at[rows_v[b].at[j]],
                                          ssem[b]).wait()

        edge_load(0)
        prep(0, 0)
        edge_load(1)

        def outer(io, _):
            for b in (0, 1):
                i = io * 2 + b
                drain(jnp.logical_and(i >= 1, trip_chunk(i - 1) < CHUNKS),
                      1 - b)
                prep(i + 1, 1 - b)
                edge_load(i + 2)
                proc(i, b)
            return 0
        lax.fori_loop(0, CPT // 2, outer, 0)
        drain(trip_chunk(CPT - 1) < CHUNKS, (CPT - 1) % 2)
        plsc.subcore_barrier()

        def emit(r0, nrows):
            pltpu.sync_copy(acc.at[pl.ds(r0, nrows)], obuf.at[pl.ds(0, nrows)])
            if not first:
                pltpu.sync_copy(x0.at[pl.ds(cN + r0, nrows)],
                                xbuf.at[pl.ds(0, nrows)])

                def mrow(r, _):
                    obuf[r, pl.ds(0, HE)] = (obuf[r, pl.ds(0, HE)] * beta
                                             + xbuf[r, pl.ds(0, HE)] * alpha)
                    return 0
                lax.fori_loop(0, nrows, mrow, 0)
            pltpu.sync_copy(obuf.at[pl.ds(0, nrows)],
                            out.at[pl.ds(cN + r0, nrows)])

        def eblk(i, _):
            ch = s + NS * i
            @pl.when(ch < RFULL)
            def _():
                emit(ch * RB, RB)
            return 0
        lax.fori_loop(0, RBPT, eblk, 0)
        @pl.when(s == (RFULL % NS))
        def _():
            emit(RFULL * RB, RTAIL)

    return pl.kernel(body,
                     out_type=jax.ShapeDtypeStruct((NC * N, HE), jnp.float32),
                     mesh=_mesh(),
                     scratch_types=scratch,
                     compiler_params=_params)


_layer_first = _make_layer(True, 1.0, 0.0)
_layer_mid = _make_layer(False, BETA, ALPHA)
_layer_last = _make_layer(False, BETA / GAMMA, ALPHA / GAMMA)


def _lookup_body(ego, users, items, u_out, i_out, idx_v, rbuf, sem):
    c = lax.axis_index("c")
    s = lax.axis_index("s")
    wid = s * NC + c
    off = wid * GPT
    for src, shift, dst, obase in (
            (users, 0, u_out, 0), (users, N, u_out, BATCH),
            (items, USERS, i_out, 0), (items, USERS + N, i_out, BATCH)):
        pltpu.sync_copy(src.at[pl.ds(off, GPT)], idx_v)
        if shift:
            for g in range(GPT // 16):
                idx_v[pl.ds(g * 16, 16)] = idx_v[pl.ds(g * 16, 16)] + shift
        pltpu.async_copy(ego.at[idx_v], rbuf, sem).wait()
        pltpu.sync_copy(rbuf, dst.at[pl.ds(obase + off, GPT)])


_lookup = pl.kernel(
    _lookup_body,
    out_type=(jax.ShapeDtypeStruct((NC * BATCH, HE), jnp.float32),
              jax.ShapeDtypeStruct((NC * BATCH, HE), jnp.float32)),
    mesh=_mesh(),
    scratch_types=[
        pltpu.VMEM((GPT,), jnp.int32),
        pltpu.VMEM((GPT, HE), jnp.float32),
        pltpu.SemaphoreType.DMA,
    ],
    compiler_params=_params)


def kernel(user_emb, item_emb, adj_vals, adj_rows, adj_cols, users, items):
    ego0 = jnp.concatenate([user_emb, item_emb], axis=0).reshape(NC * N, HE)
    vbits = lax.bitcast_convert_type(adj_vals, jnp.int32)
    edges3 = jnp.stack([adj_cols.reshape(E // SUB, SUB),
                        adj_rows.reshape(E // SUB, SUB),
                        vbits.reshape(E // SUB, SUB)], axis=1)
    h1 = _layer_first(ego0, edges3)
    h2 = _layer_mid(h1, edges3, h1)
    h3 = _layer_last(h2, edges3, h1)
    u2, i2 = _lookup(h3, users, items)
    u = jnp.concatenate([u2[:BATCH], u2[BATCH:]], axis=1)
    it = jnp.concatenate([i2[:BATCH], i2[BATCH:]], axis=1)
    return (u, it)
